# Initial kernel scaffold; baseline (speedup 1.0000x reference)
import jax
import jax.numpy as jnp
from jax.experimental import pallas as pl


def kernel(features, edge_index, edge_vals, W0, W1):
    raise NotImplementedError("write your pallas kernel here")


# speedup vs baseline: 5.8065x; 5.8065x over previous
import jax
import jax.numpy as jnp
from jax import lax
from jax.experimental import pallas as pl
from jax.experimental.pallas import tpu as pltpu
from jax.experimental.pallas import tpu_sc as plsc

N = 10000
E = 320000
D_IN = 128
D_OUT = 40
DP = 48
NC, NS, L = 2, 16, 16
NW = NC * NS
CHUNK = 128
CPW = 79
EPW = CPW * CHUNK
EP = EPW * NW
NP = 10240
RPT = NP // NS


def _mlp_body(x_ref, w0_ref, w1_ref, o_ref):
    h = jnp.maximum(
        jnp.dot(x_ref[...], w0_ref[...], preferred_element_type=jnp.float32), 0.0)
    o_ref[pl.ds(0, N), :] = jnp.dot(h, w1_ref[...],
                                    preferred_element_type=jnp.float32)


_mlp = pl.pallas_call(
    _mlp_body,
    out_shape=jax.ShapeDtypeStruct((NP, DP), jnp.float32),
)


def _mid_body(p_ref, o_ref):
    o_ref[...] = p_ref[0] + p_ref[1]


_mid = pl.pallas_call(
    _mid_body,
    out_shape=jax.ShapeDtypeStruct((NP, DP), jnp.float32),
)


def _final_body(p_ref, o_ref):
    o_ref[...] = p_ref[0, :N, :D_OUT] + p_ref[1, :N, :D_OUT]


_final = pl.pallas_call(
    _final_body,
    out_shape=jax.ShapeDtypeStruct((N, D_OUT), jnp.float32),
)


def _spmm_body(h_hbm, src_hbm, dst_hbm, vals_hbm, out_hbm,
               src_v, dst_v, vals_v, rows_v, drain_v, acc, sem):
    cid = lax.axis_index("c")
    sid = lax.axis_index("s")
    wid = sid * NC + cid

    def zrow(r, carry):
        for k in range(DP // L):
            drain_v[r, pl.ds(k * L, L)] = jnp.zeros((L,), jnp.float32)
        return carry

    lax.fori_loop(0, RPT, zrow, 0)
    pltpu.sync_copy(drain_v, acc.at[pl.ds(sid * RPT, RPT)])

    pltpu.sync_copy(src_hbm.at[wid], src_v)
    pltpu.sync_copy(dst_hbm.at[wid], dst_v)
    pltpu.sync_copy(vals_hbm.at[wid], vals_v)

    plsc.subcore_barrier()

    def chunk_body(j, carry):
        pltpu.async_copy(h_hbm.at[src_v.at[j]], rows_v, sem).wait()

        def scale(c, inner):
            v = plsc.load_gather(
                vals_v, [jnp.full((L,), j * CHUNK + c, jnp.int32)])
            for k in range(DP // L):
                seg = rows_v[c, pl.ds(k * L, L)]
                rows_v[c, pl.ds(k * L, L)] = seg * v
            return inner

        lax.fori_loop(0, CHUNK, scale, 0)

        pltpu.sync_copy(rows_v, acc.at[dst_v.at[j]], add=True)
        return carry

    lax.fori_loop(0, CPW, chunk_body, 0)

    plsc.subcore_barrier()

    pltpu.sync_copy(acc.at[pl.ds(sid * RPT, RPT)], drain_v)
    pltpu.sync_copy(drain_v, out_hbm.at[cid].at[pl.ds(sid * RPT, RPT)])


_spmm = pl.kernel(
    _spmm_body,
    out_type=jax.ShapeDtypeStruct((NC, NP, DP), jnp.float32),
    mesh=plsc.VectorSubcoreMesh(core_axis_name="c", subcore_axis_name="s"),
    compiler_params=pltpu.CompilerParams(needs_layout_passes=False,
                                         use_tc_tiling_on_sc=False),
    scratch_types=[
        pltpu.VMEM((CPW, CHUNK), jnp.int32),
        pltpu.VMEM((CPW, CHUNK), jnp.int32),
        pltpu.VMEM((EPW,), jnp.float32),
        pltpu.VMEM((CHUNK, DP), jnp.float32),
        pltpu.VMEM((RPT, DP), jnp.float32),
        pltpu.VMEM_SHARED((NP, DP), jnp.float32),
        pltpu.SemaphoreType.DMA,
    ],
)


def kernel(features, edge_index, edge_vals, W0, W1):
    w1p = jnp.pad(W1, ((0, 0), (0, DP - D_OUT)))
    h = _mlp(features, W0, w1p)

    pad = EP - E
    src = jnp.concatenate([edge_index[0], jnp.zeros((pad,), jnp.int32)])
    dst = jnp.concatenate([edge_index[1], jnp.zeros((pad,), jnp.int32)])
    vals = jnp.concatenate([edge_vals, jnp.zeros((pad,), jnp.float32)])
    src3 = src.reshape(NW, CPW, CHUNK)
    dst3 = dst.reshape(NW, CPW, CHUNK)
    vals2 = vals.reshape(NW, EPW)

    part = _spmm(h, src3, dst3, vals2)
    h = _mid(part)
    part = _spmm(h, src3, dst3, vals2)
    return _final(part)


# speedup vs baseline: 7.0856x; 1.2203x over previous
import jax
import jax.numpy as jnp
from jax import lax
from jax.experimental import pallas as pl
from jax.experimental.pallas import tpu as pltpu
from jax.experimental.pallas import tpu_sc as plsc

N = 10000
E = 320000
D_IN = 128
D_OUT = 40
DP = 48
NC, NS, L = 2, 16, 16
NW = NC * NS
CHUNK = 128
NB = 4
CPW = 80
EPW = CPW * CHUNK
EP = EPW * NW
NP = 10240
RPT = NP // NS


def _mlp_body(x_ref, w0_ref, w1_ref, o_ref):
    h = jnp.maximum(
        jnp.dot(x_ref[...], w0_ref[...], preferred_element_type=jnp.float32), 0.0)
    o_ref[pl.ds(0, N), :] = jnp.dot(h, w1_ref[...],
                                    preferred_element_type=jnp.float32)


_mlp = pl.pallas_call(
    _mlp_body,
    out_shape=jax.ShapeDtypeStruct((NP, DP), jnp.float32),
)


def _mid_body(p_ref, o_ref):
    o_ref[...] = p_ref[0] + p_ref[1]


_mid = pl.pallas_call(
    _mid_body,
    out_shape=jax.ShapeDtypeStruct((NP, DP), jnp.float32),
)


def _final_body(p_ref, o_ref):
    o_ref[...] = p_ref[0, :N, :D_OUT] + p_ref[1, :N, :D_OUT]


_final = pl.pallas_call(
    _final_body,
    out_shape=jax.ShapeDtypeStruct((N, D_OUT), jnp.float32),
)


def _spmm_body(h_hbm, src_hbm, dst_hbm, vals_hbm, out_hbm,
               src_v, dst_v, vals_v, rows_v, drain_v, acc,
               gsem, ssem):
    cid = lax.axis_index("c")
    sid = lax.axis_index("s")
    wid = sid * NC + cid

    def zrow(r, carry):
        for k in range(DP // L):
            drain_v[r, pl.ds(k * L, L)] = jnp.zeros((L,), jnp.float32)
        return carry

    lax.fori_loop(0, RPT, zrow, 0)
    pltpu.sync_copy(drain_v, acc.at[pl.ds(sid * RPT, RPT)])

    pltpu.sync_copy(src_hbm.at[wid], src_v)
    pltpu.sync_copy(dst_hbm.at[wid], dst_v)
    pltpu.sync_copy(vals_hbm.at[wid], vals_v)

    plsc.subcore_barrier()

    for b in range(NB):
        pltpu.async_copy(h_hbm.at[src_v.at[b]], rows_v.at[b], gsem[b])

    def outer(g, carry):
        for b in range(NB):
            j = g * NB + b
            pltpu.make_async_copy(
                h_hbm.at[src_v.at[j]], rows_v.at[b], gsem[b]).wait()

            @plsc.parallel_loop(0, CHUNK, 1, unroll=8)
            def _scale(c):
                v = plsc.load_gather(
                    vals_v, [jnp.full((L,), j * CHUNK + c, jnp.int32)])
                for k in range(DP // L):
                    seg = rows_v[b, c, pl.ds(k * L, L)]
                    rows_v[b, c, pl.ds(k * L, L)] = seg * v

            pltpu.async_copy(rows_v.at[b], acc.at[dst_v.at[j]], ssem[b],
                             add=True)

            b2 = (b - 1) % NB
            j2 = j - 1
            jn = j2 + NB

            @pl.when(j2 >= 0)
            def _():
                pltpu.make_async_copy(
                    rows_v.at[b2], acc.at[dst_v.at[j2]], ssem[b2]).wait()

                @pl.when(jn < CPW)
                def _():
                    pltpu.async_copy(
                        h_hbm.at[src_v.at[jn]], rows_v.at[b2], gsem[b2])
        return carry

    lax.fori_loop(0, CPW // NB, outer, 0)

    pltpu.make_async_copy(
        rows_v.at[NB - 1], acc.at[dst_v.at[CPW - 1]], ssem[NB - 1]).wait()

    plsc.subcore_barrier()

    pltpu.sync_copy(acc.at[pl.ds(sid * RPT, RPT)], drain_v)
    pltpu.sync_copy(drain_v, out_hbm.at[cid].at[pl.ds(sid * RPT, RPT)])


_spmm = pl.kernel(
    _spmm_body,
    out_type=jax.ShapeDtypeStruct((NC, NP, DP), jnp.float32),
    mesh=plsc.VectorSubcoreMesh(core_axis_name="c", subcore_axis_name="s"),
    compiler_params=pltpu.CompilerParams(needs_layout_passes=False,
                                         use_tc_tiling_on_sc=False),
    scratch_types=[
        pltpu.VMEM((CPW, CHUNK), jnp.int32),
        pltpu.VMEM((CPW, CHUNK), jnp.int32),
        pltpu.VMEM((EPW,), jnp.float32),
        pltpu.VMEM((NB, CHUNK, DP), jnp.float32),
        pltpu.VMEM((RPT, DP), jnp.float32),
        pltpu.VMEM_SHARED((NP, DP), jnp.float32),
        [pltpu.SemaphoreType.DMA] * NB,
        [pltpu.SemaphoreType.DMA] * NB,
    ],
)


def kernel(features, edge_index, edge_vals, W0, W1):
    w1p = jnp.pad(W1, ((0, 0), (0, DP - D_OUT)))
    h = _mlp(features, W0, w1p)

    pad = EP - E
    src = jnp.concatenate([edge_index[0], jnp.zeros((pad,), jnp.int32)])
    dst = jnp.concatenate([edge_index[1], jnp.zeros((pad,), jnp.int32)])
    vals = jnp.concatenate([edge_vals, jnp.zeros((pad,), jnp.float32)])
    src3 = src.reshape(NW, CPW, CHUNK)
    dst3 = dst.reshape(NW, CPW, CHUNK)
    vals2 = vals.reshape(NW, EPW)

    part = _spmm(h, src3, dst3, vals2)
    h = _mid(part)
    part = _spmm(h, src3, dst3, vals2)
    return _final(part)
